# baseline (speedup 1.0000x reference)
import functools

import jax
import jax.numpy as jnp
from jax.experimental import pallas as pl
from jax.experimental.pallas import tpu as pltpu

_BI = 1024
_BI2 = 1024
_EPS = 1e-5


def _deg_body(adj_ref, disr_ref, disc_ref):
    i = pl.program_id(1)
    bi = adj_ref.shape[1]
    s = jnp.sum(adj_ref[:], axis=-1) + 1.0
    deg = jnp.maximum(s, 1.0)
    dis = jax.lax.rsqrt(deg)
    disr_ref[:] = dis[:, :, None]
    disc_ref[0, :, pl.ds(i * bi, bi)] = dis


def _bn_stats(t2d):
    mean = jnp.mean(t2d, axis=0, keepdims=True)
    cen = t2d - mean
    var = jnp.mean(cen * cen, axis=0, keepdims=True)
    return mean, jax.lax.rsqrt(var + _EPS)


def _mm1_body(adj_ref, x_ref, w_ref, disr_ref, disc_ref, bias_ref,
              t_ref, an_ref, vb_ref):
    i = pl.program_id(1)
    bi = adj_ref.shape[1]

    @pl.when(i == 0)
    def _():
        h = jnp.dot(x_ref[0], w_ref[:],
                    preferred_element_type=jnp.float32)
        vb_ref[:] = h.astype(jnp.bfloat16)

    a = adj_ref[0]
    rows = jax.lax.broadcasted_iota(jnp.int32, a.shape, 0)
    cols = jax.lax.broadcasted_iota(jnp.int32, a.shape, 1)
    a2 = a + jnp.where(cols == rows + i * bi, 1.0, 0.0)
    dr = disr_ref[0, pl.ds(i * bi, bi), :]
    an = ((dr * a2) * disc_ref[0]).astype(jnp.bfloat16)
    an_ref[0] = an
    acc = jnp.dot(an, vb_ref[:], preferred_element_type=jnp.float32)
    t_ref[0] = acc + bias_ref[:]


def _mm23f_body(N, an_ref, t1_ref, t1b_ref, g_ref, be_ref, w_ref, bias_ref,
                out_ref, vb_ref, t2_ref, t3_ref, st_ref):
    l = pl.program_id(0)
    b = pl.program_id(1)
    i = pl.program_id(2)
    bi = an_ref.shape[1]
    C = vb_ref.shape[1]
    base = b * N + i * bi

    def prep(src2d, rows_ref):
        mean, rstd = _bn_stats(src2d)
        rows = rows_ref[pl.ds(b * N, N), :]
        yb = jnp.maximum((rows - mean) * rstd * g_ref[0] + be_ref[0], 0.0)
        vn = jnp.dot(yb.astype(jnp.bfloat16), w_ref[0].astype(jnp.bfloat16),
                     preferred_element_type=jnp.float32)
        vb_ref[:] = vn.astype(jnp.bfloat16)

    @pl.when(jnp.logical_and(l == 0, i == 0))
    def _():
        B, NN, CC = t1_ref.shape
        t1 = t1_ref[:].reshape(B * NN, CC)
        mean, rstd = _bn_stats(t1)
        yb = jnp.maximum((t1b_ref[0] - mean) * rstd * g_ref[0] + be_ref[0],
                         0.0)
        vn = jnp.dot(yb.astype(jnp.bfloat16), w_ref[0].astype(jnp.bfloat16),
                     preferred_element_type=jnp.float32)
        vb_ref[:] = vn.astype(jnp.bfloat16)

    @pl.when(jnp.logical_and(l == 1, i == 0))
    def _():
        prep(t2_ref[:], t2_ref)

    @pl.when(l < 2)
    def _():
        acc = jnp.dot(an_ref[0], vb_ref[:],
                      preferred_element_type=jnp.float32)
        t = acc + bias_ref[0]

        @pl.when(l == 0)
        def _():
            t2_ref[pl.ds(base, bi), :] = t

        @pl.when(l == 1)
        def _():
            t3_ref[pl.ds(base, bi), :] = t

    @pl.when(jnp.logical_and(jnp.logical_and(l == 2, b == 0), i == 0))
    def _():
        mean, rstd = _bn_stats(t3_ref[:])
        st_ref[0:1, :] = mean
        st_ref[1:2, :] = rstd

    @pl.when(l == 2)
    def _():
        rows = t3_ref[pl.ds(base, bi), :]
        y = jnp.maximum((rows - st_ref[0:1, :]) * st_ref[1:2, :] * g_ref[0]
                        + be_ref[0], 0.0)
        out_ref[0] = jax.nn.sigmoid(y)


def kernel(x, adj, W0, b0, g0, be0, W1, b1, g1, be1, W2, b2, g2, be2):
    B, N, _ = adj.shape
    nb = N // _BI
    C = W0.shape[1]
    f32 = jnp.float32

    disr, disc = pl.pallas_call(
        _deg_body,
        grid=(B, nb),
        in_specs=[pl.BlockSpec((1, _BI, N), lambda b, i: (b, i, 0))],
        out_specs=[
            pl.BlockSpec((1, _BI, 1), lambda b, i: (b, i, 0)),
            pl.BlockSpec((1, 1, N), lambda b, i: (b, 0, 0)),
        ],
        out_shape=[
            jax.ShapeDtypeStruct((B, N, 1), f32),
            jax.ShapeDtypeStruct((B, 1, N), f32),
        ],
    )(adj)

    row_spec = pl.BlockSpec((1, C), lambda b, i: (0, 0))
    t_spec = pl.BlockSpec((1, _BI, C), lambda b, i: (b, i, 0))
    t_shape = jax.ShapeDtypeStruct((B, N, C), f32)
    an_spec = pl.BlockSpec((1, _BI, N), lambda b, i: (b, i, 0))
    scratch = [pltpu.VMEM((N, C), jnp.bfloat16)]

    t, an = pl.pallas_call(
        _mm1_body,
        grid=(B, nb),
        in_specs=[
            pl.BlockSpec((1, _BI, N), lambda b, i: (b, i, 0)),
            pl.BlockSpec((1, N, x.shape[2]), lambda b, i: (b, 0, 0)),
            pl.BlockSpec(W0.shape, lambda b, i: (0, 0)),
            pl.BlockSpec((1, N, 1), lambda b, i: (b, 0, 0)),
            pl.BlockSpec((1, 1, N), lambda b, i: (b, 0, 0)),
            row_spec,
        ],
        out_specs=[t_spec, an_spec],
        out_shape=[t_shape, jax.ShapeDtypeStruct((B, N, N), jnp.bfloat16)],
        scratch_shapes=scratch,
    )(adj, x.astype(jnp.bfloat16), W0.astype(jnp.bfloat16), disr, disc,
      b0.reshape(1, -1))

    G = jnp.stack([g0.reshape(1, -1), g1.reshape(1, -1),
                   g2.reshape(1, -1)])
    BE = jnp.stack([be0.reshape(1, -1), be1.reshape(1, -1),
                    be2.reshape(1, -1)])
    WS = jnp.stack([W1, W2])
    BS = jnp.stack([b1.reshape(1, -1), b2.reshape(1, -1)])

    nb2 = N // _BI2

    def an_map(l, b, i):
        live = (l < 2).astype(jnp.int32)
        return (jnp.where(live, b, B - 1), jnp.where(live, i, nb2 - 1), 0)

    def t1b_map(l, b, i):
        return (jnp.where(l < 1, b, B - 1), 0, 0)

    out = pl.pallas_call(
        functools.partial(_mm23f_body, N),
        grid=(3, B, nb2),
        in_specs=[
            pl.BlockSpec((1, _BI2, N), an_map),
            pl.BlockSpec((B, N, C), lambda l, b, i: (0, 0, 0)),
            pl.BlockSpec((1, N, C), t1b_map),
            pl.BlockSpec((1, 1, C), lambda l, b, i: (l, 0, 0)),
            pl.BlockSpec((1, 1, C), lambda l, b, i: (l, 0, 0)),
            pl.BlockSpec((1, C, C),
                         lambda l, b, i: (jnp.minimum(l, 1), 0, 0)),
            pl.BlockSpec((1, 1, C),
                         lambda l, b, i: (jnp.minimum(l, 1), 0, 0)),
        ],
        out_specs=pl.BlockSpec((1, _BI2, C), lambda l, b, i: (b, i, 0)),
        out_shape=jax.ShapeDtypeStruct((B, N, C), f32),
        scratch_shapes=[
            pltpu.VMEM((N, C), jnp.bfloat16),
            pltpu.VMEM((B * N, C), f32),
            pltpu.VMEM((B * N, C), f32),
            pltpu.VMEM((2, C), f32),
        ],
    )(an, t, t, G, BE, WS, BS)
    return out


# speedup vs baseline: 1.0075x; 1.0075x over previous
import functools

import jax
import jax.numpy as jnp
from jax.experimental import pallas as pl
from jax.experimental.pallas import tpu as pltpu

_BI = 1024
_BI2 = 1024
_EPS = 1e-5


def _degmm1_body(N, adj_ref, x_ref, w_ref, bias_ref, t_ref, an_ref,
                 vb_ref, disr_ref, disc_ref):
    l = pl.program_id(0)
    b = pl.program_id(1)
    i = pl.program_id(2)
    bi = adj_ref.shape[1]
    base = b * N + i * bi

    @pl.when(l == 0)
    def _():
        s = jnp.sum(adj_ref[:], axis=-1) + 1.0
        deg = jnp.maximum(s, 1.0)
        dis = jax.lax.rsqrt(deg)
        disr_ref[pl.ds(base, bi), :] = dis[:, :, None][0]
        disc_ref[0:1, pl.ds(base, bi)] = dis

    @pl.when(jnp.logical_and(l == 1, i == 0))
    def _():
        h = jnp.dot(x_ref[0], w_ref[:],
                    preferred_element_type=jnp.float32)
        vb_ref[:] = h.astype(jnp.bfloat16)

    @pl.when(l == 1)
    def _():
        a = adj_ref[0]
        rows = jax.lax.broadcasted_iota(jnp.int32, a.shape, 0)
        cols = jax.lax.broadcasted_iota(jnp.int32, a.shape, 1)
        a2 = a + jnp.where(cols == rows + i * bi, 1.0, 0.0)
        dr = disr_ref[pl.ds(base, bi), :]
        dc = disc_ref[0:1, pl.ds(b * N, N)]
        an = ((dr * a2) * dc).astype(jnp.bfloat16)
        an_ref[0] = an
        acc = jnp.dot(an, vb_ref[:], preferred_element_type=jnp.float32)
        t_ref[0] = acc + bias_ref[:]


def _bn_stats(t2d):
    mean = jnp.mean(t2d, axis=0, keepdims=True)
    cen = t2d - mean
    var = jnp.mean(cen * cen, axis=0, keepdims=True)
    return mean, jax.lax.rsqrt(var + _EPS)


def _mm23f_body(N, an_ref, t1_ref, t1b_ref, g_ref, be_ref, w_ref, bias_ref,
                out_ref, vb_ref, t2_ref, t3_ref, st_ref):
    l = pl.program_id(0)
    b = pl.program_id(1)
    i = pl.program_id(2)
    bi = an_ref.shape[1]
    C = vb_ref.shape[1]
    base = b * N + i * bi

    def prep(src2d, rows_ref):
        mean, rstd = _bn_stats(src2d)
        rows = rows_ref[pl.ds(b * N, N), :]
        yb = jnp.maximum((rows - mean) * rstd * g_ref[0] + be_ref[0], 0.0)
        vn = jnp.dot(yb.astype(jnp.bfloat16), w_ref[0].astype(jnp.bfloat16),
                     preferred_element_type=jnp.float32)
        vb_ref[:] = vn.astype(jnp.bfloat16)

    @pl.when(jnp.logical_and(l == 0, i == 0))
    def _():
        B, NN, CC = t1_ref.shape
        t1 = t1_ref[:].reshape(B * NN, CC)
        mean, rstd = _bn_stats(t1)
        yb = jnp.maximum((t1b_ref[0] - mean) * rstd * g_ref[0] + be_ref[0],
                         0.0)
        vn = jnp.dot(yb.astype(jnp.bfloat16), w_ref[0].astype(jnp.bfloat16),
                     preferred_element_type=jnp.float32)
        vb_ref[:] = vn.astype(jnp.bfloat16)

    @pl.when(jnp.logical_and(l == 1, i == 0))
    def _():
        prep(t2_ref[:], t2_ref)

    @pl.when(l < 2)
    def _():
        acc = jnp.dot(an_ref[0], vb_ref[:],
                      preferred_element_type=jnp.float32)
        t = acc + bias_ref[0]

        @pl.when(l == 0)
        def _():
            t2_ref[pl.ds(base, bi), :] = t

        @pl.when(l == 1)
        def _():
            t3_ref[pl.ds(base, bi), :] = t

    @pl.when(jnp.logical_and(jnp.logical_and(l == 2, b == 0), i == 0))
    def _():
        mean, rstd = _bn_stats(t3_ref[:])
        st_ref[0:1, :] = mean
        st_ref[1:2, :] = rstd

    @pl.when(l == 2)
    def _():
        rows = t3_ref[pl.ds(base, bi), :]
        y = jnp.maximum((rows - st_ref[0:1, :]) * st_ref[1:2, :] * g_ref[0]
                        + be_ref[0], 0.0)
        out_ref[0] = jax.nn.sigmoid(y)


def kernel(x, adj, W0, b0, g0, be0, W1, b1, g1, be1, W2, b2, g2, be2):
    B, N, _ = adj.shape
    nb = N // _BI
    C = W0.shape[1]
    f32 = jnp.float32

    def freeze01(l, b, i):
        live = (l == 1).astype(jnp.int32)
        return (jnp.where(live, b, 0), jnp.where(live, i, 0), 0)

    t, an = pl.pallas_call(
        functools.partial(_degmm1_body, N),
        grid=(2, B, nb),
        in_specs=[
            pl.BlockSpec((1, _BI, N), lambda l, b, i: (b, i, 0)),
            pl.BlockSpec((1, N, x.shape[2]),
                         lambda l, b, i: (jnp.where(l == 1, b, 0), 0, 0)),
            pl.BlockSpec(W0.shape, lambda l, b, i: (0, 0)),
            pl.BlockSpec((1, C), lambda l, b, i: (0, 0)),
        ],
        out_specs=[
            pl.BlockSpec((1, _BI, C), freeze01),
            pl.BlockSpec((1, _BI, N), freeze01),
        ],
        out_shape=[
            jax.ShapeDtypeStruct((B, N, C), f32),
            jax.ShapeDtypeStruct((B, N, N), jnp.bfloat16),
        ],
        scratch_shapes=[
            pltpu.VMEM((N, C), jnp.bfloat16),
            pltpu.VMEM((B * N, 1), f32),
            pltpu.VMEM((1, B * N), f32),
        ],
    )(adj, x.astype(jnp.bfloat16), W0.astype(jnp.bfloat16),
      b0.reshape(1, -1))

    G = jnp.stack([g0.reshape(1, -1), g1.reshape(1, -1),
                   g2.reshape(1, -1)])
    BE = jnp.stack([be0.reshape(1, -1), be1.reshape(1, -1),
                    be2.reshape(1, -1)])
    WS = jnp.stack([W1, W2])
    BS = jnp.stack([b1.reshape(1, -1), b2.reshape(1, -1)])

    nb2 = N // _BI2

    def an_map(l, b, i):
        live = (l < 2).astype(jnp.int32)
        return (jnp.where(live, b, B - 1), jnp.where(live, i, nb2 - 1), 0)

    def t1b_map(l, b, i):
        return (jnp.where(l < 1, b, B - 1), 0, 0)

    out = pl.pallas_call(
        functools.partial(_mm23f_body, N),
        grid=(3, B, nb2),
        in_specs=[
            pl.BlockSpec((1, _BI2, N), an_map),
            pl.BlockSpec((B, N, C), lambda l, b, i: (0, 0, 0)),
            pl.BlockSpec((1, N, C), t1b_map),
            pl.BlockSpec((1, 1, C), lambda l, b, i: (l, 0, 0)),
            pl.BlockSpec((1, 1, C), lambda l, b, i: (l, 0, 0)),
            pl.BlockSpec((1, C, C),
                         lambda l, b, i: (jnp.minimum(l, 1), 0, 0)),
            pl.BlockSpec((1, 1, C),
                         lambda l, b, i: (jnp.minimum(l, 1), 0, 0)),
        ],
        out_specs=pl.BlockSpec((1, _BI2, C), lambda l, b, i: (b, i, 0)),
        out_shape=jax.ShapeDtypeStruct((B, N, C), f32),
        scratch_shapes=[
            pltpu.VMEM((N, C), jnp.bfloat16),
            pltpu.VMEM((B * N, C), f32),
            pltpu.VMEM((B * N, C), f32),
            pltpu.VMEM((2, C), f32),
        ],
    )(an, t, t, G, BE, WS, BS)
    return out
